# baseline (speedup 1.0000x reference)
import jax
import jax.numpy as jnp
from jax import lax
from jax.experimental import pallas as pl
from jax.experimental.pallas import tpu as pltpu
from jax.experimental.pallas import tpu_sc as plsc

EMBED = 32
PACK = 4
SUPER = EMBED * PACK
BATCH = 16384
NC = 2
NS = 16
NW = NC * NS
BPW = BATCH // NW
CHUNK = 128
NCHUNK = BPW // CHUNK
LANES = 16


def _tt_kernel(pos_h, pos_t, pos_r, pos_tem,
               neg_h, neg_t, neg_r, neg_tem,
               ent_w, rel_w, tem_w,
               pos_out, neg_out,
               idx_h, idx_t, idx_r, idx_tm,
               sidx_h, sidx_t, sidx_r, sidx_tm,
               rows_h, rows_t, rows_r, rows_tm,
               out_v, sem):
    wid = lax.axis_index("s") * NC + lax.axis_index("c")
    base = wid * BPW
    iota = lax.iota(jnp.int32, LANES)

    def do_side(ih, it, ir, itm, out_hbm):
        pltpu.sync_copy(ih.at[pl.ds(base, BPW)], idx_h)
        pltpu.sync_copy(it.at[pl.ds(base, BPW)], idx_t)
        pltpu.sync_copy(ir.at[pl.ds(base, BPW)], idx_r)
        pltpu.sync_copy(itm.at[pl.ds(base, BPW)], idx_tm)

        def sbody(v, carry):
            sl = pl.ds(v * LANES, LANES)
            sidx_h[sl] = idx_h[sl] >> 2
            sidx_t[sl] = idx_t[sl] >> 2
            sidx_r[sl] = idx_r[sl] >> 2
            sidx_tm[sl] = idx_tm[sl] >> 2
            return carry

        lax.fori_loop(0, BPW // LANES, sbody, 0)

        for c in range(NCHUNK):
            sl = pl.ds(c * CHUNK, CHUNK)
            cps = [
                pltpu.async_copy(ent_w.at[sidx_h.at[sl]], rows_h, sem),
                pltpu.async_copy(ent_w.at[sidx_t.at[sl]], rows_t, sem),
                pltpu.async_copy(rel_w.at[sidx_r.at[sl]], rows_r, sem),
                pltpu.async_copy(tem_w.at[sidx_tm.at[sl]], rows_tm, sem),
            ]
            for cp in cps:
                cp.wait()

            def gbody(g, carry):
                row = g * LANES + iota
                bsl = pl.ds(c * CHUNK + g * LANES, LANES)
                off_h = (idx_h[bsl] & 3) << 5
                off_t = (idx_t[bsl] & 3) << 5
                off_r = (idx_r[bsl] & 3) << 5
                off_tm = (idx_tm[bsl] & 3) << 5
                s = jnp.zeros((LANES,), jnp.float32)
                for j in range(EMBED):
                    vh = plsc.load_gather(rows_h, [row, off_h + j])
                    vt = plsc.load_gather(rows_t, [row, off_t + j])
                    vr = plsc.load_gather(rows_r, [row, off_r + j])
                    vtm = plsc.load_gather(rows_tm, [row, off_tm + j])
                    s = s + jnp.abs(vh + vr + vtm - vt)
                out_v[bsl] = s
                return carry

            lax.fori_loop(0, CHUNK // LANES, gbody, 0)

        pltpu.sync_copy(out_v, out_hbm.at[pl.ds(base, BPW)])

    do_side(pos_h, pos_t, pos_r, pos_tem, pos_out)
    do_side(neg_h, neg_t, neg_r, neg_tem, neg_out)


def kernel(pos_h, pos_t, pos_r, pos_tem, neg_h, neg_t, neg_r, neg_tem,
           ent_w, rel_w, tem_w):
    mesh = plsc.VectorSubcoreMesh(core_axis_name="c", subcore_axis_name="s")
    f = pl.kernel(
        _tt_kernel,
        mesh=mesh,
        out_type=(
            jax.ShapeDtypeStruct((BATCH,), jnp.float32),
            jax.ShapeDtypeStruct((BATCH,), jnp.float32),
        ),
        scratch_types=[
            pltpu.VMEM((BPW,), jnp.int32),
            pltpu.VMEM((BPW,), jnp.int32),
            pltpu.VMEM((BPW,), jnp.int32),
            pltpu.VMEM((BPW,), jnp.int32),
            pltpu.VMEM((BPW,), jnp.int32),
            pltpu.VMEM((BPW,), jnp.int32),
            pltpu.VMEM((BPW,), jnp.int32),
            pltpu.VMEM((BPW,), jnp.int32),
            pltpu.VMEM((CHUNK, SUPER), jnp.float32),
            pltpu.VMEM((CHUNK, SUPER), jnp.float32),
            pltpu.VMEM((CHUNK, SUPER), jnp.float32),
            pltpu.VMEM((CHUNK, SUPER), jnp.float32),
            pltpu.VMEM((BPW,), jnp.float32),
            pltpu.SemaphoreType.DMA,
        ],
        compiler_params=pltpu.CompilerParams(needs_layout_passes=False),
    )
    i32 = jnp.int32
    return f(pos_h.astype(i32), pos_t.astype(i32), pos_r.astype(i32),
             pos_tem.astype(i32), neg_h.astype(i32), neg_t.astype(i32),
             neg_r.astype(i32), neg_tem.astype(i32),
             ent_w.reshape(-1, SUPER), rel_w.reshape(-1, SUPER),
             tem_w.reshape(-1, SUPER))
